# baseline (speedup 1.0000x reference)
import functools

import jax
import jax.numpy as jnp
from jax import lax
from jax.experimental import pallas as pl
from jax.experimental.pallas import tpu as pltpu
from jax.experimental.pallas import tpu_sc as plsc

N = 10000
E = 160000
NC, NS = 2, 16
FW = 128

ZROWS = 48
WPT = 624

K1 = 64
EPT1 = E // NS
IRT1 = 160
RB1 = 40
NCH1 = IRT1 // RB1

K2 = 64
EPT2 = E // (NC * NS)
IRT2 = 80
RB2 = 40
NCH2 = IRT2 // RB2

_F32 = jnp.float32
_I32 = jnp.int32


def _degree_call(edge2):
    mesh = plsc.VectorSubcoreMesh(core_axis_name="c", subcore_axis_name="s")
    CH = 2000
    EPT = E // NS

    @functools.partial(
        pl.kernel,
        out_type=jax.ShapeDtypeStruct((NC * NS, N), _F32),
        mesh=mesh,
        compiler_params=pltpu.CompilerParams(needs_layout_passes=False),
        scratch_types=[
            pltpu.VMEM((CH,), _I32),
            pltpu.VMEM((N,), _F32),
        ],
    )
    def deg_kernel(edge_hbm, out_hbm, idx_v, hist_v):
        c = lax.axis_index("c")
        s = lax.axis_index("s")

        def zero_body(i, carry):
            hist_v[pl.ds(i * 16, 16)] = jnp.zeros((16,), _F32)
            return carry

        lax.fori_loop(0, N // 16, zero_body, 0)

        base = c * E + s * EPT
        ones = jnp.ones((16,), _F32)

        def chunk_body(k, carry):
            pltpu.sync_copy(edge_hbm.at[pl.ds(base + k * CH, CH)], idx_v)

            def inner(j, carry2):
                iv = idx_v[pl.ds(j * 16, 16)]
                plsc.addupdate_scatter(hist_v, [iv], ones)
                return carry2

            lax.fori_loop(0, CH // 16, inner, 0)
            return carry

        lax.fori_loop(0, EPT // CH, chunk_body, 0)
        pltpu.sync_copy(hist_v, out_hbm.at[c * NS + s])

    return deg_kernel(edge2)


def _agg_call(m2, src_rows, dst_rows, zblk, *, k, irt, rb, nchunk):
    mesh = plsc.VectorSubcoreMesh(core_axis_name="c", subcore_axis_name="s")

    @functools.partial(
        pl.kernel,
        out_type=jax.ShapeDtypeStruct((NC * N, FW), _F32),
        mesh=mesh,
        compiler_params=pltpu.CompilerParams(needs_layout_passes=False),
        scratch_types=[
            pltpu.VMEM_SHARED((N + 16, FW), _F32),
            pltpu.VMEM((rb, k), _I32),
            pltpu.VMEM((rb, k), _I32),
            pltpu.VMEM((k, FW), _F32),
            pltpu.VMEM((k, FW), _F32),
            pltpu.VMEM((k, FW), _F32),
            pltpu.VMEM((k, FW), _F32),
            pltpu.VMEM((ZROWS, FW), _F32),
            pltpu.SemaphoreType.DMA,
            pltpu.SemaphoreType.DMA,
            pltpu.SemaphoreType.DMA,
            pltpu.SemaphoreType.DMA,
            pltpu.SemaphoreType.DMA,
            pltpu.SemaphoreType.DMA,
            pltpu.SemaphoreType.DMA,
            pltpu.SemaphoreType.DMA,
            pltpu.SemaphoreType.DMA,
        ],
    )
    def agg_kernel(m_hbm, src_hbm, dst_hbm, zero_hbm, out_hbm,
                   acc, idx_s, idx_d, rows0, rows1, rows2, rows3, zbuf,
                   gs0, gs1, gs2, gs3, ss0, ss1, ss2, ss3, zsem):
        c = lax.axis_index("c")
        s = lax.axis_index("s")

        zscope = jax.named_scope("agg_zero")
        zscope.__enter__()
        pltpu.sync_copy(zero_hbm, zbuf)
        for z in range(WPT // ZROWS):
            pltpu.async_copy(zbuf, acc.at[pl.ds(s * WPT + z * ZROWS, ZROWS)], zsem)

        @pl.when(s == NS - 1)
        def _zero_tail():
            pltpu.sync_copy(zbuf.at[pl.ds(0, 16)], acc.at[pl.ds(NS * WPT, 16)])

        for z in range(WPT // ZROWS):
            pltpu.make_async_copy(
                zbuf, acc.at[pl.ds(s * WPT + z * ZROWS, ZROWS)], zsem).wait()

        plsc.subcore_barrier()
        zscope.__exit__(None, None, None)

        escope = jax.named_scope("agg_edges")
        escope.__enter__()
        slab = (c * NS + s) * irt
        bufs = (rows0, rows1, rows2, rows3)
        gsems = (gs0, gs1, gs2, gs3)
        ssems = (ss0, ss1, ss2, ss3)

        def chunk_body(kk, carry):
            pltpu.sync_copy(src_hbm.at[pl.ds(slab + kk * rb, rb)], idx_s)
            pltpu.sync_copy(dst_hbm.at[pl.ds(slab + kk * rb, rb)], idx_d)

            def gath(j):
                pltpu.async_copy(m_hbm.at[idx_s.at[j]], bufs[j % 4], gsems[j % 4])

            def gath_wait(j):
                pltpu.make_async_copy(
                    m_hbm.at[idx_s.at[j]], bufs[j % 4], gsems[j % 4]).wait()

            def scat(j):
                pltpu.async_copy(
                    bufs[j % 4], acc.at[idx_d.at[j]], ssems[j % 4], add=True)

            def scat_wait(j):
                pltpu.make_async_copy(
                    bufs[j % 4], acc.at[idx_d.at[j]], ssems[j % 4]).wait()

            gath(0)
            gath(1)
            gath(2)
            for j in range(rb):
                gath_wait(j)
                scat(j)
                if j + 3 < rb:
                    if j >= 1:
                        scat_wait(j - 1)
                    gath(j + 3)
            scat_wait(rb - 4)
            scat_wait(rb - 3)
            scat_wait(rb - 2)
            scat_wait(rb - 1)
            return carry

        lax.fori_loop(0, nchunk, chunk_body, 0)
        plsc.subcore_barrier()
        escope.__exit__(None, None, None)
        wscope = jax.named_scope("agg_writeback")
        wscope.__enter__()
        for z in range(WPT // ZROWS):
            pltpu.async_copy(acc.at[pl.ds(s * WPT + z * ZROWS, ZROWS)],
                             out_hbm.at[pl.ds(c * N + s * WPT + z * ZROWS, ZROWS)],
                             zsem)

        @pl.when(s == NS - 1)
        def _write_tail():
            pltpu.sync_copy(acc.at[pl.ds(NS * WPT, 16)],
                            out_hbm.at[pl.ds(c * N + NS * WPT, 16)])

        for z in range(WPT // ZROWS):
            pltpu.make_async_copy(
                acc.at[pl.ds(s * WPT + z * ZROWS, ZROWS)],
                out_hbm.at[pl.ds(c * N + s * WPT + z * ZROWS, ZROWS)], zsem).wait()
        wscope.__exit__(None, None, None)

    return agg_kernel(m2, src_rows, dst_rows, zblk)


_B = 1000


def _tc0_body(degt_ref, x_ref, w_ref, m1_ref, no_ref, ni_ref):
    dp = degt_ref[...]
    d_o = jnp.sum(dp[:, :NS], axis=1, keepdims=True)
    d_i = jnp.sum(dp[:, NS:], axis=1, keepdims=True)
    n_o = lax.rsqrt(jnp.where(d_o > 0, d_o, 1.0))
    n_i = lax.rsqrt(jnp.where(d_i > 0, d_i, 1.0))
    no_ref[...] = n_o
    ni_ref[...] = n_i
    m = jnp.dot(x_ref[...], w_ref[...], preferred_element_type=_F32)
    m = m * n_o
    m1_ref[0] = m[:, :FW]
    m1_ref[1] = m[:, FW:]


def _tc0_call(degT, X, W1):
    return pl.pallas_call(
        _tc0_body,
        grid=(N // _B,),
        in_specs=[
            pl.BlockSpec((_B, NC * NS), lambda i: (i, 0)),
            pl.BlockSpec((_B, 256), lambda i: (i, 0)),
            pl.BlockSpec((256, 256), lambda i: (0, 0)),
        ],
        out_specs=[
            pl.BlockSpec((2, _B, FW), lambda i: (0, i, 0)),
            pl.BlockSpec((_B, 1), lambda i: (i, 0)),
            pl.BlockSpec((_B, 1), lambda i: (i, 0)),
        ],
        out_shape=[
            jax.ShapeDtypeStruct((2, N, FW), _F32),
            jax.ShapeDtypeStruct((N, 1), _F32),
            jax.ShapeDtypeStruct((N, 1), _F32),
        ],
    )(degT, X, W1)


def _tc1_body(a_ref, ni_ref, no_ref, b_ref, w_ref, out_ref):
    ni = ni_ref[...]
    no = no_ref[...]
    h0 = a_ref[0] * ni + b_ref[0]
    h0 = jnp.where(h0 > 0, h0, 0.1 * h0)
    h1 = a_ref[1] * ni + b_ref[1]
    h1 = jnp.where(h1 > 0, h1, 0.1 * h1)
    m = (jnp.dot(h0, w_ref[0], preferred_element_type=_F32)
         + jnp.dot(h1, w_ref[1], preferred_element_type=_F32))
    m = m * no
    out_ref[0] = m
    out_ref[1] = m


def _tc1_call(agg1, n_i, n_o, b1h, w2h):
    return pl.pallas_call(
        _tc1_body,
        grid=(N // _B,),
        in_specs=[
            pl.BlockSpec((2, _B, FW), lambda i: (0, i, 0)),
            pl.BlockSpec((_B, 1), lambda i: (i, 0)),
            pl.BlockSpec((_B, 1), lambda i: (i, 0)),
            pl.BlockSpec((2, 1, FW), lambda i: (0, 0, 0)),
            pl.BlockSpec((2, FW, FW), lambda i: (0, 0, 0)),
        ],
        out_specs=pl.BlockSpec((2, _B, FW), lambda i: (0, i, 0)),
        out_shape=jax.ShapeDtypeStruct((2, N, FW), _F32),
    )(agg1, n_i, n_o, b1h, w2h)


def _tc2_body(a_ref, ni_ref, no_ref, b_ref, w_ref, out_ref):
    ni = ni_ref[...]
    no = no_ref[...]
    a = a_ref[0] + a_ref[1]
    h = a * ni + b_ref[...]
    h = jnp.where(h > 0, h, 0.1 * h)
    m = jnp.dot(h, w_ref[...], preferred_element_type=_F32) * no
    mp = jnp.concatenate([m, jnp.zeros((_B, FW - 64), _F32)], axis=1)
    out_ref[0] = mp
    out_ref[1] = mp


def _tc2_call(agg2, n_i, n_o, b2r, W3):
    return pl.pallas_call(
        _tc2_body,
        grid=(N // _B,),
        in_specs=[
            pl.BlockSpec((2, _B, FW), lambda i: (0, i, 0)),
            pl.BlockSpec((_B, 1), lambda i: (i, 0)),
            pl.BlockSpec((_B, 1), lambda i: (i, 0)),
            pl.BlockSpec((1, FW), lambda i: (0, 0)),
            pl.BlockSpec((FW, 64), lambda i: (0, 0)),
        ],
        out_specs=pl.BlockSpec((2, _B, FW), lambda i: (0, i, 0)),
        out_shape=jax.ShapeDtypeStruct((2, N, FW), _F32),
    )(agg2, n_i, n_o, b2r, W3)


def _tc3_body(a_ref, ni_ref, b_ref, st_ref, out_ref):
    ni = ni_ref[...]
    a = (a_ref[0] + a_ref[1])[:, :64]
    h = a * ni + b_ref[...]
    logits = jnp.dot(h, st_ref[...], preferred_element_type=_F32)
    mx = jnp.max(logits, axis=1, keepdims=True)
    e = jnp.exp(logits - mx)
    out_ref[...] = e / jnp.sum(e, axis=1, keepdims=True)


def _tc3_call(agg3, n_i, b3r, st):
    ncls = st.shape[1]
    return pl.pallas_call(
        _tc3_body,
        grid=(N // _B,),
        in_specs=[
            pl.BlockSpec((2, _B, FW), lambda i: (0, i, 0)),
            pl.BlockSpec((_B, 1), lambda i: (i, 0)),
            pl.BlockSpec((1, 64), lambda i: (0, 0)),
            pl.BlockSpec((64, ncls), lambda i: (0, 0)),
        ],
        out_specs=pl.BlockSpec((_B, ncls), lambda i: (i, 0)),
        out_shape=jax.ShapeDtypeStruct((N, ncls), _F32),
    )(agg3, n_i, b3r, st)


def kernel(X, edge_index, S, W1, b1, W2, b2, W3, b3):
    ei = edge_index.astype(_I32)
    src = ei[0]
    dst = ei[1]

    edge2 = jnp.concatenate([src, dst])

    npad1 = IRT1 * K1 - EPT1
    s3 = src.reshape(NS, EPT1)
    s3p = jnp.concatenate(
        [s3, jnp.zeros((NS, npad1), _I32)], axis=1)
    src1 = jnp.concatenate([s3p, s3p + N], axis=0).reshape(NC * NS * IRT1, K1)
    d3 = dst.reshape(NS, EPT1)
    d3p = jnp.concatenate(
        [d3, jnp.full((NS, npad1), N, _I32)], axis=1)
    dst1 = jnp.concatenate([d3p, d3p], axis=0).reshape(NC * NS * IRT1, K1)

    npad2 = IRT2 * K2 - EPT2
    s4 = src.reshape(NC * NS, EPT2)
    s4p = jnp.concatenate(
        [s4, jnp.zeros((NC * NS, npad2), _I32)], axis=1)
    src2 = s4p.reshape(NC * NS * IRT2, K2)
    d4 = dst.reshape(NC * NS, EPT2)
    d4p = jnp.concatenate(
        [d4, jnp.full((NC * NS, npad2), N, _I32)], axis=1)
    dst2 = d4p.reshape(NC * NS * IRT2, K2)
    core_off = jnp.repeat(jnp.arange(NC, dtype=_I32) * N, NS * IRT2)
    src2 = src2 + core_off[:, None]

    zblk = jnp.zeros((ZROWS, FW), _F32)

    degp = _degree_call(edge2)
    degT = degp.T

    M1, n_o, n_i = _tc0_call(degT, X, W1)

    agg1 = _agg_call(M1.reshape(2 * N, FW), src1, dst1, zblk,
                     k=K1, irt=IRT1, rb=RB1, nchunk=NCH1)
    M2 = _tc1_call(agg1.reshape(2, N, FW), n_i, n_o,
                   b1.reshape(2, 1, FW), W2.reshape(2, FW, FW))

    agg2 = _agg_call(M2.reshape(2 * N, FW), src2, dst2, zblk,
                     k=K2, irt=IRT2, rb=RB2, nchunk=NCH2)
    M3 = _tc2_call(agg2.reshape(2, N, FW), n_i, n_o, b2.reshape(1, FW), W3)

    agg3 = _agg_call(M3.reshape(2 * N, FW), src2, dst2, zblk,
                     k=K2, irt=IRT2, rb=RB2, nchunk=NCH2)
    return _tc3_call(agg3.reshape(2, N, FW), n_i, b3.reshape(1, 64), S.T)


# speedup vs baseline: 1.0083x; 1.0083x over previous
import functools

import jax
import jax.numpy as jnp
from jax import lax
from jax.experimental import pallas as pl
from jax.experimental.pallas import tpu as pltpu
from jax.experimental.pallas import tpu_sc as plsc

N = 10000
E = 160000
NC, NS = 2, 16
FW = 128

ZROWS = 48
WPT = 624

K1 = 80
EPT1 = E // NS
IRT1 = 128
RB1 = 32
NCH1 = IRT1 // RB1

K2 = 80
EPT2 = E // (NC * NS)
IRT2 = 64
RB2 = 32
NCH2 = IRT2 // RB2

_F32 = jnp.float32
_I32 = jnp.int32


def _degree_call(edge2):
    mesh = plsc.VectorSubcoreMesh(core_axis_name="c", subcore_axis_name="s")
    CH = 2000
    EPT = E // NS

    @functools.partial(
        pl.kernel,
        out_type=jax.ShapeDtypeStruct((NC * NS, N), _F32),
        mesh=mesh,
        compiler_params=pltpu.CompilerParams(needs_layout_passes=False),
        scratch_types=[
            pltpu.VMEM((CH,), _I32),
            pltpu.VMEM((N,), _F32),
        ],
    )
    def deg_kernel(edge_hbm, out_hbm, idx_v, hist_v):
        c = lax.axis_index("c")
        s = lax.axis_index("s")

        def zero_body(i, carry):
            hist_v[pl.ds(i * 16, 16)] = jnp.zeros((16,), _F32)
            return carry

        lax.fori_loop(0, N // 16, zero_body, 0)

        base = c * E + s * EPT
        ones = jnp.ones((16,), _F32)

        def chunk_body(k, carry):
            pltpu.sync_copy(edge_hbm.at[pl.ds(base + k * CH, CH)], idx_v)

            def inner(j, carry2):
                iv = idx_v[pl.ds(j * 16, 16)]
                plsc.addupdate_scatter(hist_v, [iv], ones)
                return carry2

            lax.fori_loop(0, CH // 16, inner, 0)
            return carry

        lax.fori_loop(0, EPT // CH, chunk_body, 0)
        pltpu.sync_copy(hist_v, out_hbm.at[c * NS + s])

    return deg_kernel(edge2)


def _agg_call(m2, src_rows, dst_rows, zblk, *, k, irt, rb, nchunk):
    mesh = plsc.VectorSubcoreMesh(core_axis_name="c", subcore_axis_name="s")

    @functools.partial(
        pl.kernel,
        out_type=jax.ShapeDtypeStruct((NC * N, FW), _F32),
        mesh=mesh,
        compiler_params=pltpu.CompilerParams(needs_layout_passes=False),
        scratch_types=[
            pltpu.VMEM_SHARED((N + 16, FW), _F32),
            pltpu.VMEM((rb, k), _I32),
            pltpu.VMEM((rb, k), _I32),
            pltpu.VMEM((k, FW), _F32),
            pltpu.VMEM((k, FW), _F32),
            pltpu.VMEM((k, FW), _F32),
            pltpu.VMEM((ZROWS, FW), _F32),
            pltpu.SemaphoreType.DMA,
            pltpu.SemaphoreType.DMA,
            pltpu.SemaphoreType.DMA,
            pltpu.SemaphoreType.DMA,
            pltpu.SemaphoreType.DMA,
            pltpu.SemaphoreType.DMA,
            pltpu.SemaphoreType.DMA,
        ],
    )
    def agg_kernel(m_hbm, src_hbm, dst_hbm, zero_hbm, out_hbm,
                   acc, idx_s, idx_d, rows0, rows1, rows2, zbuf,
                   gs0, gs1, gs2, ss0, ss1, ss2, zsem):
        c = lax.axis_index("c")
        s = lax.axis_index("s")

        zscope = jax.named_scope("agg_zero")
        zscope.__enter__()
        pltpu.sync_copy(zero_hbm, zbuf)
        for z in range(WPT // ZROWS):
            pltpu.async_copy(zbuf, acc.at[pl.ds(s * WPT + z * ZROWS, ZROWS)], zsem)

        @pl.when(s == NS - 1)
        def _zero_tail():
            pltpu.sync_copy(zbuf.at[pl.ds(0, 16)], acc.at[pl.ds(NS * WPT, 16)])

        for z in range(WPT // ZROWS):
            pltpu.make_async_copy(
                zbuf, acc.at[pl.ds(s * WPT + z * ZROWS, ZROWS)], zsem).wait()

        plsc.subcore_barrier()
        zscope.__exit__(None, None, None)

        escope = jax.named_scope("agg_edges")
        escope.__enter__()
        slab = (c * NS + s) * irt
        bufs = (rows0, rows1, rows2)
        gsems = (gs0, gs1, gs2)
        ssems = (ss0, ss1, ss2)

        def chunk_body(kk, carry):
            pltpu.sync_copy(src_hbm.at[pl.ds(slab + kk * rb, rb)], idx_s)
            pltpu.sync_copy(dst_hbm.at[pl.ds(slab + kk * rb, rb)], idx_d)

            def gath(j):
                pltpu.async_copy(m_hbm.at[idx_s.at[j]], bufs[j % 3], gsems[j % 3])

            def gath_wait(j):
                pltpu.make_async_copy(
                    m_hbm.at[idx_s.at[j]], bufs[j % 3], gsems[j % 3]).wait()

            def scat(j):
                pltpu.async_copy(
                    bufs[j % 3], acc.at[idx_d.at[j]], ssems[j % 3], add=True)

            def scat_wait(j):
                pltpu.make_async_copy(
                    bufs[j % 3], acc.at[idx_d.at[j]], ssems[j % 3]).wait()

            gath(0)
            gath(1)
            for j in range(rb):
                gath_wait(j)
                scat(j)
                if j + 2 < rb:
                    if j >= 1:
                        scat_wait(j - 1)
                    gath(j + 2)
            scat_wait(rb - 3)
            scat_wait(rb - 2)
            scat_wait(rb - 1)
            return carry

        lax.fori_loop(0, nchunk, chunk_body, 0)
        plsc.subcore_barrier()
        escope.__exit__(None, None, None)
        wscope = jax.named_scope("agg_writeback")
        wscope.__enter__()
        for z in range(WPT // ZROWS):
            pltpu.async_copy(acc.at[pl.ds(s * WPT + z * ZROWS, ZROWS)],
                             out_hbm.at[pl.ds(c * N + s * WPT + z * ZROWS, ZROWS)],
                             zsem)

        @pl.when(s == NS - 1)
        def _write_tail():
            pltpu.sync_copy(acc.at[pl.ds(NS * WPT, 16)],
                            out_hbm.at[pl.ds(c * N + NS * WPT, 16)])

        for z in range(WPT // ZROWS):
            pltpu.make_async_copy(
                acc.at[pl.ds(s * WPT + z * ZROWS, ZROWS)],
                out_hbm.at[pl.ds(c * N + s * WPT + z * ZROWS, ZROWS)], zsem).wait()
        wscope.__exit__(None, None, None)

    return agg_kernel(m2, src_rows, dst_rows, zblk)


_B = 1000


def _tc0_body(degt_ref, x_ref, w_ref, m1_ref, no_ref, ni_ref):
    dp = degt_ref[...]
    d_o = jnp.sum(dp[:, :NS], axis=1, keepdims=True)
    d_i = jnp.sum(dp[:, NS:], axis=1, keepdims=True)
    n_o = lax.rsqrt(jnp.where(d_o > 0, d_o, 1.0))
    n_i = lax.rsqrt(jnp.where(d_i > 0, d_i, 1.0))
    no_ref[...] = n_o
    ni_ref[...] = n_i
    m = jnp.dot(x_ref[...], w_ref[...], preferred_element_type=_F32)
    m = m * n_o
    m1_ref[0] = m[:, :FW]
    m1_ref[1] = m[:, FW:]


def _tc0_call(degT, X, W1):
    return pl.pallas_call(
        _tc0_body,
        grid=(N // _B,),
        in_specs=[
            pl.BlockSpec((_B, NC * NS), lambda i: (i, 0)),
            pl.BlockSpec((_B, 256), lambda i: (i, 0)),
            pl.BlockSpec((256, 256), lambda i: (0, 0)),
        ],
        out_specs=[
            pl.BlockSpec((2, _B, FW), lambda i: (0, i, 0)),
            pl.BlockSpec((_B, 1), lambda i: (i, 0)),
            pl.BlockSpec((_B, 1), lambda i: (i, 0)),
        ],
        out_shape=[
            jax.ShapeDtypeStruct((2, N, FW), _F32),
            jax.ShapeDtypeStruct((N, 1), _F32),
            jax.ShapeDtypeStruct((N, 1), _F32),
        ],
    )(degT, X, W1)


def _tc1_body(a_ref, ni_ref, no_ref, b_ref, w_ref, out_ref):
    ni = ni_ref[...]
    no = no_ref[...]
    h0 = a_ref[0] * ni + b_ref[0]
    h0 = jnp.where(h0 > 0, h0, 0.1 * h0)
    h1 = a_ref[1] * ni + b_ref[1]
    h1 = jnp.where(h1 > 0, h1, 0.1 * h1)
    m = (jnp.dot(h0, w_ref[0], preferred_element_type=_F32)
         + jnp.dot(h1, w_ref[1], preferred_element_type=_F32))
    m = m * no
    out_ref[0] = m
    out_ref[1] = m


def _tc1_call(agg1, n_i, n_o, b1h, w2h):
    return pl.pallas_call(
        _tc1_body,
        grid=(N // _B,),
        in_specs=[
            pl.BlockSpec((2, _B, FW), lambda i: (0, i, 0)),
            pl.BlockSpec((_B, 1), lambda i: (i, 0)),
            pl.BlockSpec((_B, 1), lambda i: (i, 0)),
            pl.BlockSpec((2, 1, FW), lambda i: (0, 0, 0)),
            pl.BlockSpec((2, FW, FW), lambda i: (0, 0, 0)),
        ],
        out_specs=pl.BlockSpec((2, _B, FW), lambda i: (0, i, 0)),
        out_shape=jax.ShapeDtypeStruct((2, N, FW), _F32),
    )(agg1, n_i, n_o, b1h, w2h)


def _tc2_body(a_ref, ni_ref, no_ref, b_ref, w_ref, out_ref):
    ni = ni_ref[...]
    no = no_ref[...]
    a = a_ref[0] + a_ref[1]
    h = a * ni + b_ref[...]
    h = jnp.where(h > 0, h, 0.1 * h)
    m = jnp.dot(h, w_ref[...], preferred_element_type=_F32) * no
    mp = jnp.concatenate([m, jnp.zeros((_B, FW - 64), _F32)], axis=1)
    out_ref[0] = mp
    out_ref[1] = mp


def _tc2_call(agg2, n_i, n_o, b2r, W3):
    return pl.pallas_call(
        _tc2_body,
        grid=(N // _B,),
        in_specs=[
            pl.BlockSpec((2, _B, FW), lambda i: (0, i, 0)),
            pl.BlockSpec((_B, 1), lambda i: (i, 0)),
            pl.BlockSpec((_B, 1), lambda i: (i, 0)),
            pl.BlockSpec((1, FW), lambda i: (0, 0)),
            pl.BlockSpec((FW, 64), lambda i: (0, 0)),
        ],
        out_specs=pl.BlockSpec((2, _B, FW), lambda i: (0, i, 0)),
        out_shape=jax.ShapeDtypeStruct((2, N, FW), _F32),
    )(agg2, n_i, n_o, b2r, W3)


def _tc3_body(a_ref, ni_ref, b_ref, st_ref, out_ref):
    ni = ni_ref[...]
    a = (a_ref[0] + a_ref[1])[:, :64]
    h = a * ni + b_ref[...]
    logits = jnp.dot(h, st_ref[...], preferred_element_type=_F32)
    mx = jnp.max(logits, axis=1, keepdims=True)
    e = jnp.exp(logits - mx)
    out_ref[...] = e / jnp.sum(e, axis=1, keepdims=True)


def _tc3_call(agg3, n_i, b3r, st):
    ncls = st.shape[1]
    return pl.pallas_call(
        _tc3_body,
        grid=(N // _B,),
        in_specs=[
            pl.BlockSpec((2, _B, FW), lambda i: (0, i, 0)),
            pl.BlockSpec((_B, 1), lambda i: (i, 0)),
            pl.BlockSpec((1, 64), lambda i: (0, 0)),
            pl.BlockSpec((64, ncls), lambda i: (0, 0)),
        ],
        out_specs=pl.BlockSpec((_B, ncls), lambda i: (i, 0)),
        out_shape=jax.ShapeDtypeStruct((N, ncls), _F32),
    )(agg3, n_i, b3r, st)


def kernel(X, edge_index, S, W1, b1, W2, b2, W3, b3):
    ei = edge_index.astype(_I32)
    src = ei[0]
    dst = ei[1]

    edge2 = jnp.concatenate([src, dst])

    npad1 = IRT1 * K1 - EPT1
    s3 = src.reshape(NS, EPT1)
    s3p = jnp.concatenate(
        [s3, jnp.zeros((NS, npad1), _I32)], axis=1)
    src1 = jnp.concatenate([s3p, s3p + N], axis=0).reshape(NC * NS * IRT1, K1)
    d3 = dst.reshape(NS, EPT1)
    d3p = jnp.concatenate(
        [d3, jnp.full((NS, npad1), N, _I32)], axis=1)
    dst1 = jnp.concatenate([d3p, d3p], axis=0).reshape(NC * NS * IRT1, K1)

    npad2 = IRT2 * K2 - EPT2
    s4 = src.reshape(NC * NS, EPT2)
    s4p = jnp.concatenate(
        [s4, jnp.zeros((NC * NS, npad2), _I32)], axis=1)
    src2 = s4p.reshape(NC * NS * IRT2, K2)
    d4 = dst.reshape(NC * NS, EPT2)
    d4p = jnp.concatenate(
        [d4, jnp.full((NC * NS, npad2), N, _I32)], axis=1)
    dst2 = d4p.reshape(NC * NS * IRT2, K2)
    core_off = jnp.repeat(jnp.arange(NC, dtype=_I32) * N, NS * IRT2)
    src2 = src2 + core_off[:, None]

    zblk = jnp.zeros((ZROWS, FW), _F32)

    degp = _degree_call(edge2)
    degT = degp.T

    M1, n_o, n_i = _tc0_call(degT, X, W1)

    agg1 = _agg_call(M1.reshape(2 * N, FW), src1, dst1, zblk,
                     k=K1, irt=IRT1, rb=RB1, nchunk=NCH1)
    M2 = _tc1_call(agg1.reshape(2, N, FW), n_i, n_o,
                   b1.reshape(2, 1, FW), W2.reshape(2, FW, FW))

    agg2 = _agg_call(M2.reshape(2 * N, FW), src2, dst2, zblk,
                     k=K2, irt=IRT2, rb=RB2, nchunk=NCH2)
    M3 = _tc2_call(agg2.reshape(2, N, FW), n_i, n_o, b2.reshape(1, FW), W3)

    agg3 = _agg_call(M3.reshape(2 * N, FW), src2, dst2, zblk,
                     k=K2, irt=IRT2, rb=RB2, nchunk=NCH2)
    return _tc3_call(agg3.reshape(2, N, FW), n_i, b3.reshape(1, 64), S.T)


# speedup vs baseline: 1.0232x; 1.0147x over previous
import functools

import jax
import jax.numpy as jnp
from jax import lax
from jax.experimental import pallas as pl
from jax.experimental.pallas import tpu as pltpu
from jax.experimental.pallas import tpu_sc as plsc

N = 10000
E = 160000
NC, NS = 2, 16
FW = 128

ZROWS = 48
WPT = 624

K1 = 80
EPT1 = E // NS
IRT1 = 128
RB1 = 32
NCH1 = IRT1 // RB1

K2 = 80
EPT2 = E // (NC * NS)
IRT2 = 64
RB2 = 32
NCH2 = IRT2 // RB2

_F32 = jnp.float32
_I32 = jnp.int32


def _degree_call(edge2):
    mesh = plsc.VectorSubcoreMesh(core_axis_name="c", subcore_axis_name="s")
    CH = 2000
    EPT = E // NS

    @functools.partial(
        pl.kernel,
        out_type=jax.ShapeDtypeStruct((NC * NS, N), _F32),
        mesh=mesh,
        compiler_params=pltpu.CompilerParams(needs_layout_passes=False),
        scratch_types=[
            pltpu.VMEM((CH,), _I32),
            pltpu.VMEM((N,), _F32),
        ],
    )
    def deg_kernel(edge_hbm, out_hbm, idx_v, hist_v):
        c = lax.axis_index("c")
        s = lax.axis_index("s")

        def zero_body(i, carry):
            hist_v[pl.ds(i * 16, 16)] = jnp.zeros((16,), _F32)
            return carry

        lax.fori_loop(0, N // 16, zero_body, 0)

        base = c * E + s * EPT
        ones = jnp.ones((16,), _F32)

        def chunk_body(k, carry):
            pltpu.sync_copy(edge_hbm.at[pl.ds(base + k * CH, CH)], idx_v)

            def inner(j, carry2):
                iv = idx_v[pl.ds(j * 16, 16)]
                plsc.addupdate_scatter(hist_v, [iv], ones)
                return carry2

            lax.fori_loop(0, CH // 16, inner, 0)
            return carry

        lax.fori_loop(0, EPT // CH, chunk_body, 0)
        pltpu.sync_copy(hist_v, out_hbm.at[c * NS + s])

    return deg_kernel(edge2)


def _agg_call(m2, src_rows, dst_rows, zblk, *, k, irt, rb, nchunk):
    mesh = plsc.VectorSubcoreMesh(core_axis_name="c", subcore_axis_name="s")

    @functools.partial(
        pl.kernel,
        out_type=jax.ShapeDtypeStruct((NC * N, FW), _F32),
        mesh=mesh,
        compiler_params=pltpu.CompilerParams(needs_layout_passes=False),
        scratch_types=[
            pltpu.VMEM_SHARED((N + 16, FW), _F32),
            pltpu.VMEM((rb, k), _I32),
            pltpu.VMEM((rb, k), _I32),
            pltpu.VMEM((k, FW), _F32),
            pltpu.VMEM((k, FW), _F32),
            pltpu.VMEM((k, FW), _F32),
            pltpu.VMEM((ZROWS, FW), _F32),
            pltpu.SemaphoreType.DMA,
            pltpu.SemaphoreType.DMA,
            pltpu.SemaphoreType.DMA,
            pltpu.SemaphoreType.DMA,
            pltpu.SemaphoreType.DMA,
            pltpu.SemaphoreType.DMA,
            pltpu.SemaphoreType.DMA,
        ],
    )
    def agg_kernel(m_hbm, src_hbm, dst_hbm, zero_hbm, out_hbm,
                   acc, idx_s, idx_d, rows0, rows1, rows2, zbuf,
                   gs0, gs1, gs2, ss0, ss1, ss2, zsem):
        c = lax.axis_index("c")
        s = lax.axis_index("s")

        zscope = jax.named_scope("agg_zero")
        zscope.__enter__()
        pltpu.sync_copy(zero_hbm, zbuf)
        for z in range(WPT // ZROWS):
            pltpu.async_copy(zbuf, acc.at[pl.ds(s * WPT + z * ZROWS, ZROWS)], zsem)

        @pl.when(s == NS - 1)
        def _zero_tail():
            pltpu.sync_copy(zbuf.at[pl.ds(0, 16)], acc.at[pl.ds(NS * WPT, 16)])

        for z in range(WPT // ZROWS):
            pltpu.make_async_copy(
                zbuf, acc.at[pl.ds(s * WPT + z * ZROWS, ZROWS)], zsem).wait()

        plsc.subcore_barrier()
        zscope.__exit__(None, None, None)

        escope = jax.named_scope("agg_edges")
        escope.__enter__()
        slab = (c * NS + s) * irt
        bufs = (rows0, rows1, rows2)
        gsems = (gs0, gs1, gs2)
        ssems = (ss0, ss1, ss2)

        def chunk_body(kk, carry):
            pltpu.sync_copy(src_hbm.at[pl.ds(slab + kk * rb, rb)], idx_s)
            pltpu.sync_copy(dst_hbm.at[pl.ds(slab + kk * rb, rb)], idx_d)

            def gath(j):
                pltpu.async_copy(m_hbm.at[idx_s.at[j]], bufs[j % 3], gsems[j % 3])

            def gath_wait(j):
                pltpu.make_async_copy(
                    m_hbm.at[idx_s.at[j]], bufs[j % 3], gsems[j % 3]).wait()

            def scat(j):
                pltpu.async_copy(
                    bufs[j % 3], acc.at[idx_d.at[j]], ssems[j % 3], add=True)

            def scat_wait(j):
                pltpu.make_async_copy(
                    bufs[j % 3], acc.at[idx_d.at[j]], ssems[j % 3]).wait()

            gath(0)
            gath(1)
            for j in range(rb):
                gath_wait(j)
                scat(j)
                if j + 2 < rb:
                    if j >= 1:
                        scat_wait(j - 1)
                    gath(j + 2)
            scat_wait(rb - 3)
            scat_wait(rb - 2)
            scat_wait(rb - 1)
            return carry

        lax.fori_loop(0, nchunk, chunk_body, 0)
        plsc.subcore_barrier()
        escope.__exit__(None, None, None)
        wscope = jax.named_scope("agg_writeback")
        wscope.__enter__()
        for z in range(WPT // ZROWS):
            pltpu.async_copy(acc.at[pl.ds(s * WPT + z * ZROWS, ZROWS)],
                             out_hbm.at[pl.ds(c * N + s * WPT + z * ZROWS, ZROWS)],
                             zsem)

        @pl.when(s == NS - 1)
        def _write_tail():
            pltpu.sync_copy(acc.at[pl.ds(NS * WPT, 16)],
                            out_hbm.at[pl.ds(c * N + NS * WPT, 16)])

        for z in range(WPT // ZROWS):
            pltpu.make_async_copy(
                acc.at[pl.ds(s * WPT + z * ZROWS, ZROWS)],
                out_hbm.at[pl.ds(c * N + s * WPT + z * ZROWS, ZROWS)], zsem).wait()
        wscope.__exit__(None, None, None)

    return agg_kernel(m2, src_rows, dst_rows, zblk)


_B = 1000


def _tc0_body(degp_ref, x_ref, w_ref, m1_ref, no_ref, ni_ref):
    dp = degp_ref[...]
    d_o = jnp.sum(dp[:, :NS], axis=1, keepdims=True)
    d_i = jnp.sum(dp[:, NS:], axis=1, keepdims=True)
    n_o = lax.rsqrt(jnp.where(d_o > 0, d_o, 1.0))
    n_i = lax.rsqrt(jnp.where(d_i > 0, d_i, 1.0))
    no_ref[...] = n_o
    ni_ref[...] = n_i
    m = jnp.dot(x_ref[...], w_ref[...], preferred_element_type=_F32)
    m = m * n_o
    m1_ref[0] = m[:, :FW]
    m1_ref[1] = m[:, FW:]


def _tc0_call(degp, X, W1):
    return pl.pallas_call(
        _tc0_body,
        grid=(N // _B,),
        in_specs=[
            pl.BlockSpec((_B, NC * NS), lambda i: (i, 0)),
            pl.BlockSpec((_B, 256), lambda i: (i, 0)),
            pl.BlockSpec((256, 256), lambda i: (0, 0)),
        ],
        out_specs=[
            pl.BlockSpec((2, _B, FW), lambda i: (0, i, 0)),
            pl.BlockSpec((_B, 1), lambda i: (i, 0)),
            pl.BlockSpec((_B, 1), lambda i: (i, 0)),
        ],
        out_shape=[
            jax.ShapeDtypeStruct((2, N, FW), _F32),
            jax.ShapeDtypeStruct((N, 1), _F32),
            jax.ShapeDtypeStruct((N, 1), _F32),
        ],
    )(degp, X, W1)


def _tc1_body(a_ref, ni_ref, no_ref, b_ref, w_ref, out_ref):
    ni = ni_ref[...]
    no = no_ref[...]
    h0 = a_ref[0] * ni + b_ref[0]
    h0 = jnp.where(h0 > 0, h0, 0.1 * h0)
    h1 = a_ref[1] * ni + b_ref[1]
    h1 = jnp.where(h1 > 0, h1, 0.1 * h1)
    m = (jnp.dot(h0, w_ref[0], preferred_element_type=_F32)
         + jnp.dot(h1, w_ref[1], preferred_element_type=_F32))
    m = m * no
    out_ref[0] = m
    out_ref[1] = m


def _tc1_call(agg1, n_i, n_o, b1h, w2h):
    return pl.pallas_call(
        _tc1_body,
        grid=(N // _B,),
        in_specs=[
            pl.BlockSpec((2, _B, FW), lambda i: (0, i, 0)),
            pl.BlockSpec((_B, 1), lambda i: (i, 0)),
            pl.BlockSpec((_B, 1), lambda i: (i, 0)),
            pl.BlockSpec((2, 1, FW), lambda i: (0, 0, 0)),
            pl.BlockSpec((2, FW, FW), lambda i: (0, 0, 0)),
        ],
        out_specs=pl.BlockSpec((2, _B, FW), lambda i: (0, i, 0)),
        out_shape=jax.ShapeDtypeStruct((2, N, FW), _F32),
    )(agg1, n_i, n_o, b1h, w2h)


def _tc2_body(a_ref, ni_ref, no_ref, b_ref, w_ref, out_ref):
    ni = ni_ref[...]
    no = no_ref[...]
    a = a_ref[0] + a_ref[1]
    h = a * ni + b_ref[...]
    h = jnp.where(h > 0, h, 0.1 * h)
    m = jnp.dot(h, w_ref[...], preferred_element_type=_F32) * no
    mp = jnp.concatenate([m, jnp.zeros((_B, FW - 64), _F32)], axis=1)
    out_ref[0] = mp
    out_ref[1] = mp


def _tc2_call(agg2, n_i, n_o, b2r, W3):
    return pl.pallas_call(
        _tc2_body,
        grid=(N // _B,),
        in_specs=[
            pl.BlockSpec((2, _B, FW), lambda i: (0, i, 0)),
            pl.BlockSpec((_B, 1), lambda i: (i, 0)),
            pl.BlockSpec((_B, 1), lambda i: (i, 0)),
            pl.BlockSpec((1, FW), lambda i: (0, 0)),
            pl.BlockSpec((FW, 64), lambda i: (0, 0)),
        ],
        out_specs=pl.BlockSpec((2, _B, FW), lambda i: (0, i, 0)),
        out_shape=jax.ShapeDtypeStruct((2, N, FW), _F32),
    )(agg2, n_i, n_o, b2r, W3)


def _tc3_body(a_ref, ni_ref, b_ref, st_ref, out_ref):
    ni = ni_ref[...]
    a = (a_ref[0] + a_ref[1])[:, :64]
    h = a * ni + b_ref[...]
    logits = jnp.dot(h, st_ref[...], preferred_element_type=_F32)
    mx = jnp.max(logits, axis=1, keepdims=True)
    e = jnp.exp(logits - mx)
    out_ref[...] = e / jnp.sum(e, axis=1, keepdims=True)


def _tc3_call(agg3, n_i, b3r, st):
    ncls = st.shape[1]
    return pl.pallas_call(
        _tc3_body,
        grid=(N // _B,),
        in_specs=[
            pl.BlockSpec((2, _B, FW), lambda i: (0, i, 0)),
            pl.BlockSpec((_B, 1), lambda i: (i, 0)),
            pl.BlockSpec((1, 64), lambda i: (0, 0)),
            pl.BlockSpec((64, ncls), lambda i: (0, 0)),
        ],
        out_specs=pl.BlockSpec((_B, ncls), lambda i: (i, 0)),
        out_shape=jax.ShapeDtypeStruct((N, ncls), _F32),
    )(agg3, n_i, b3r, st)


def kernel(X, edge_index, S, W1, b1, W2, b2, W3, b3):
    ei = edge_index.astype(_I32)
    src = ei[0]
    dst = ei[1]

    edge2 = ei.reshape(2 * E)

    npad1 = IRT1 * K1 - EPT1
    s3 = src.reshape(NS, EPT1)
    s3p = jnp.concatenate(
        [s3, jnp.zeros((NS, npad1), _I32)], axis=1)
    src1 = jnp.concatenate([s3p, s3p + N], axis=0).reshape(NC * NS * IRT1, K1)
    d3 = dst.reshape(NS, EPT1)
    d3p = jnp.concatenate(
        [d3, jnp.full((NS, npad1), N, _I32)], axis=1)
    dst1 = jnp.concatenate([d3p, d3p], axis=0).reshape(NC * NS * IRT1, K1)

    npad2 = IRT2 * K2 - EPT2
    s4 = src.reshape(NC * NS, EPT2)
    s4p = jnp.concatenate(
        [s4, jnp.zeros((NC * NS, npad2), _I32)], axis=1)
    src2 = s4p.reshape(NC * NS * IRT2, K2)
    d4 = dst.reshape(NC * NS, EPT2)
    d4p = jnp.concatenate(
        [d4, jnp.full((NC * NS, npad2), N, _I32)], axis=1)
    dst2 = d4p.reshape(NC * NS * IRT2, K2)
    core_off = jnp.repeat(jnp.arange(NC, dtype=_I32) * N, NS * IRT2)
    src2 = src2 + core_off[:, None]

    zblk = jnp.zeros((ZROWS, FW), _F32)

    degp = _degree_call(edge2)

    M1, n_o, n_i = _tc0_call(degp.T, X, W1)

    agg1 = _agg_call(M1.reshape(2 * N, FW), src1, dst1, zblk,
                     k=K1, irt=IRT1, rb=RB1, nchunk=NCH1)
    M2 = _tc1_call(agg1.reshape(2, N, FW), n_i, n_o,
                   b1.reshape(2, 1, FW), W2.reshape(2, FW, FW))

    agg2 = _agg_call(M2.reshape(2 * N, FW), src2, dst2, zblk,
                     k=K2, irt=IRT2, rb=RB2, nchunk=NCH2)
    M3 = _tc2_call(agg2.reshape(2, N, FW), n_i, n_o, b2.reshape(1, FW), W3)

    agg3 = _agg_call(M3.reshape(2 * N, FW), src2, dst2, zblk,
                     k=K2, irt=IRT2, rb=RB2, nchunk=NCH2)
    return _tc3_call(agg3.reshape(2, N, FW), n_i, b3.reshape(1, 64), S.T)


# speedup vs baseline: 1.0235x; 1.0003x over previous
import functools

import jax
import jax.numpy as jnp
from jax import lax
from jax.experimental import pallas as pl
from jax.experimental.pallas import tpu as pltpu
from jax.experimental.pallas import tpu_sc as plsc

N = 10000
E = 160000
NC, NS = 2, 16
FW = 128

ZROWS = 48
WPT = 624

K1 = 80
EPT1 = E // NS
IRT1 = 128
RB1 = 32
NCH1 = IRT1 // RB1

K2 = 80
EPT2 = E // (NC * NS)
IRT2 = 64
RB2 = 32
NCH2 = IRT2 // RB2

_F32 = jnp.float32
_I32 = jnp.int32


def _degree_call(edge2):
    mesh = plsc.VectorSubcoreMesh(core_axis_name="c", subcore_axis_name="s")
    CH = 2000
    EPT = E // NS

    @functools.partial(
        pl.kernel,
        out_type=jax.ShapeDtypeStruct((NC * NS, N), _F32),
        mesh=mesh,
        compiler_params=pltpu.CompilerParams(needs_layout_passes=False),
        scratch_types=[
            pltpu.VMEM((CH,), _I32),
            pltpu.VMEM((N,), _F32),
        ],
    )
    def deg_kernel(edge_hbm, out_hbm, idx_v, hist_v):
        c = lax.axis_index("c")
        s = lax.axis_index("s")

        def zero_body(i, carry):
            hist_v[pl.ds(i * 16, 16)] = jnp.zeros((16,), _F32)
            return carry

        lax.fori_loop(0, N // 16, zero_body, 0)

        base = c * E + s * EPT
        ones = jnp.ones((16,), _F32)

        def chunk_body(k, carry):
            pltpu.sync_copy(edge_hbm.at[pl.ds(base + k * CH, CH)], idx_v)

            def inner(j, carry2):
                iv = idx_v[pl.ds(j * 16, 16)]
                plsc.addupdate_scatter(hist_v, [iv], ones)
                return carry2

            lax.fori_loop(0, CH // 16, inner, 0)
            return carry

        lax.fori_loop(0, EPT // CH, chunk_body, 0)
        pltpu.sync_copy(hist_v, out_hbm.at[c * NS + s])

    return deg_kernel(edge2)


def _agg_call(m2, src_rows, dst_rows, zblk, *, k, irt, rb, nchunk):
    mesh = plsc.VectorSubcoreMesh(core_axis_name="c", subcore_axis_name="s")

    @functools.partial(
        pl.kernel,
        out_type=jax.ShapeDtypeStruct((NC * N, FW), _F32),
        mesh=mesh,
        compiler_params=pltpu.CompilerParams(needs_layout_passes=False),
        scratch_types=[
            pltpu.VMEM_SHARED((N + 16, FW), _F32),
            pltpu.VMEM((rb, k), _I32),
            pltpu.VMEM((rb, k), _I32),
            pltpu.VMEM((k, FW), _F32),
            pltpu.VMEM((k, FW), _F32),
            pltpu.VMEM((k, FW), _F32),
            pltpu.VMEM((ZROWS, FW), _F32),
            pltpu.SemaphoreType.DMA,
            pltpu.SemaphoreType.DMA,
            pltpu.SemaphoreType.DMA,
            pltpu.SemaphoreType.DMA,
            pltpu.SemaphoreType.DMA,
            pltpu.SemaphoreType.DMA,
            pltpu.SemaphoreType.DMA,
        ],
    )
    def agg_kernel(m_hbm, src_hbm, dst_hbm, zero_hbm, out_hbm,
                   acc, idx_s, idx_d, rows0, rows1, rows2, zbuf,
                   gs0, gs1, gs2, ss0, ss1, ss2, zsem):
        c = lax.axis_index("c")
        s = lax.axis_index("s")

        pltpu.sync_copy(zero_hbm, zbuf)
        for z in range(WPT // ZROWS):
            pltpu.async_copy(zbuf, acc.at[pl.ds(s * WPT + z * ZROWS, ZROWS)], zsem)

        @pl.when(s == NS - 1)
        def _zero_tail():
            pltpu.sync_copy(zbuf.at[pl.ds(0, 16)], acc.at[pl.ds(NS * WPT, 16)])

        for z in range(WPT // ZROWS):
            pltpu.make_async_copy(
                zbuf, acc.at[pl.ds(s * WPT + z * ZROWS, ZROWS)], zsem).wait()

        plsc.subcore_barrier()

        slab = (c * NS + s) * irt
        bufs = (rows0, rows1, rows2)
        gsems = (gs0, gs1, gs2)
        ssems = (ss0, ss1, ss2)

        def chunk_body(kk, carry):
            pltpu.sync_copy(src_hbm.at[pl.ds(slab + kk * rb, rb)], idx_s)
            pltpu.sync_copy(dst_hbm.at[pl.ds(slab + kk * rb, rb)], idx_d)

            def gath(j):
                pltpu.async_copy(m_hbm.at[idx_s.at[j]], bufs[j % 3], gsems[j % 3])

            def gath_wait(j):
                pltpu.make_async_copy(
                    m_hbm.at[idx_s.at[j]], bufs[j % 3], gsems[j % 3]).wait()

            def scat(j):
                pltpu.async_copy(
                    bufs[j % 3], acc.at[idx_d.at[j]], ssems[j % 3], add=True)

            def scat_wait(j):
                pltpu.make_async_copy(
                    bufs[j % 3], acc.at[idx_d.at[j]], ssems[j % 3]).wait()

            gath(0)
            gath(1)
            for j in range(rb):
                gath_wait(j)
                scat(j)
                if j + 2 < rb:
                    if j >= 1:
                        scat_wait(j - 1)
                    gath(j + 2)
            scat_wait(rb - 3)
            scat_wait(rb - 2)
            scat_wait(rb - 1)
            return carry

        lax.fori_loop(0, nchunk, chunk_body, 0)
        plsc.subcore_barrier()
        for z in range(WPT // ZROWS):
            pltpu.async_copy(acc.at[pl.ds(s * WPT + z * ZROWS, ZROWS)],
                             out_hbm.at[pl.ds(c * N + s * WPT + z * ZROWS, ZROWS)],
                             zsem)

        @pl.when(s == NS - 1)
        def _write_tail():
            pltpu.sync_copy(acc.at[pl.ds(NS * WPT, 16)],
                            out_hbm.at[pl.ds(c * N + NS * WPT, 16)])

        for z in range(WPT // ZROWS):
            pltpu.make_async_copy(
                acc.at[pl.ds(s * WPT + z * ZROWS, ZROWS)],
                out_hbm.at[pl.ds(c * N + s * WPT + z * ZROWS, ZROWS)], zsem).wait()

    return agg_kernel(m2, src_rows, dst_rows, zblk)


_B = 1000


def _tc0_body(degp_ref, x_ref, w_ref, m1_ref, no_ref, ni_ref):
    dp = degp_ref[...]
    d_o = jnp.sum(dp[:, :NS], axis=1, keepdims=True)
    d_i = jnp.sum(dp[:, NS:], axis=1, keepdims=True)
    n_o = lax.rsqrt(jnp.where(d_o > 0, d_o, 1.0))
    n_i = lax.rsqrt(jnp.where(d_i > 0, d_i, 1.0))
    no_ref[...] = n_o
    ni_ref[...] = n_i
    m = jnp.dot(x_ref[...], w_ref[...], preferred_element_type=_F32)
    m = m * n_o
    m1_ref[0] = m[:, :FW]
    m1_ref[1] = m[:, FW:]


def _tc0_call(degp, X, W1):
    return pl.pallas_call(
        _tc0_body,
        grid=(N // _B,),
        in_specs=[
            pl.BlockSpec((_B, NC * NS), lambda i: (i, 0)),
            pl.BlockSpec((_B, 256), lambda i: (i, 0)),
            pl.BlockSpec((256, 256), lambda i: (0, 0)),
        ],
        out_specs=[
            pl.BlockSpec((2, _B, FW), lambda i: (0, i, 0)),
            pl.BlockSpec((_B, 1), lambda i: (i, 0)),
            pl.BlockSpec((_B, 1), lambda i: (i, 0)),
        ],
        out_shape=[
            jax.ShapeDtypeStruct((2, N, FW), _F32),
            jax.ShapeDtypeStruct((N, 1), _F32),
            jax.ShapeDtypeStruct((N, 1), _F32),
        ],
    )(degp, X, W1)


def _tc1_body(a_ref, ni_ref, no_ref, b_ref, w_ref, out_ref):
    ni = ni_ref[...]
    no = no_ref[...]
    h0 = a_ref[0] * ni + b_ref[0]
    h0 = jnp.where(h0 > 0, h0, 0.1 * h0)
    h1 = a_ref[1] * ni + b_ref[1]
    h1 = jnp.where(h1 > 0, h1, 0.1 * h1)
    m = (jnp.dot(h0, w_ref[0], preferred_element_type=_F32)
         + jnp.dot(h1, w_ref[1], preferred_element_type=_F32))
    m = m * no
    out_ref[0] = m
    out_ref[1] = m


def _tc1_call(agg1, n_i, n_o, b1h, w2h):
    return pl.pallas_call(
        _tc1_body,
        grid=(N // _B,),
        in_specs=[
            pl.BlockSpec((2, _B, FW), lambda i: (0, i, 0)),
            pl.BlockSpec((_B, 1), lambda i: (i, 0)),
            pl.BlockSpec((_B, 1), lambda i: (i, 0)),
            pl.BlockSpec((2, 1, FW), lambda i: (0, 0, 0)),
            pl.BlockSpec((2, FW, FW), lambda i: (0, 0, 0)),
        ],
        out_specs=pl.BlockSpec((2, _B, FW), lambda i: (0, i, 0)),
        out_shape=jax.ShapeDtypeStruct((2, N, FW), _F32),
    )(agg1, n_i, n_o, b1h, w2h)


def _tc2_body(a_ref, ni_ref, no_ref, b_ref, w_ref, out_ref):
    ni = ni_ref[...]
    no = no_ref[...]
    a = a_ref[0] + a_ref[1]
    h = a * ni + b_ref[...]
    h = jnp.where(h > 0, h, 0.1 * h)
    m = jnp.dot(h, w_ref[...], preferred_element_type=_F32) * no
    mp = jnp.concatenate([m, jnp.zeros((_B, FW - 64), _F32)], axis=1)
    out_ref[0] = mp
    out_ref[1] = mp


def _tc2_call(agg2, n_i, n_o, b2r, W3):
    return pl.pallas_call(
        _tc2_body,
        grid=(N // _B,),
        in_specs=[
            pl.BlockSpec((2, _B, FW), lambda i: (0, i, 0)),
            pl.BlockSpec((_B, 1), lambda i: (i, 0)),
            pl.BlockSpec((_B, 1), lambda i: (i, 0)),
            pl.BlockSpec((1, FW), lambda i: (0, 0)),
            pl.BlockSpec((FW, 64), lambda i: (0, 0)),
        ],
        out_specs=pl.BlockSpec((2, _B, FW), lambda i: (0, i, 0)),
        out_shape=jax.ShapeDtypeStruct((2, N, FW), _F32),
    )(agg2, n_i, n_o, b2r, W3)


def _tc3_body(a_ref, ni_ref, b_ref, st_ref, out_ref):
    ni = ni_ref[...]
    a = (a_ref[0] + a_ref[1])[:, :64]
    h = a * ni + b_ref[...]
    logits = jnp.dot(h, st_ref[...], preferred_element_type=_F32)
    mx = jnp.max(logits, axis=1, keepdims=True)
    e = jnp.exp(logits - mx)
    out_ref[...] = e / jnp.sum(e, axis=1, keepdims=True)


def _tc3_call(agg3, n_i, b3r, st):
    ncls = st.shape[1]
    return pl.pallas_call(
        _tc3_body,
        grid=(N // _B,),
        in_specs=[
            pl.BlockSpec((2, _B, FW), lambda i: (0, i, 0)),
            pl.BlockSpec((_B, 1), lambda i: (i, 0)),
            pl.BlockSpec((1, 64), lambda i: (0, 0)),
            pl.BlockSpec((64, ncls), lambda i: (0, 0)),
        ],
        out_specs=pl.BlockSpec((_B, ncls), lambda i: (i, 0)),
        out_shape=jax.ShapeDtypeStruct((N, ncls), _F32),
    )(agg3, n_i, b3r, st)


def kernel(X, edge_index, S, W1, b1, W2, b2, W3, b3):
    ei = edge_index.astype(_I32)
    src = ei[0]
    dst = ei[1]

    edge2 = ei.reshape(2 * E)

    npad1 = IRT1 * K1 - EPT1
    s3 = src.reshape(NS, EPT1)
    s3p = jnp.concatenate(
        [s3, jnp.zeros((NS, npad1), _I32)], axis=1)
    src1 = jnp.concatenate([s3p, s3p + N], axis=0).reshape(NC * NS * IRT1, K1)
    d3 = dst.reshape(NS, EPT1)
    d3p = jnp.concatenate(
        [d3, jnp.full((NS, npad1), N, _I32)], axis=1)
    dst1 = jnp.concatenate([d3p, d3p], axis=0).reshape(NC * NS * IRT1, K1)

    npad2 = IRT2 * K2 - EPT2
    s4 = src.reshape(NC * NS, EPT2)
    s4p = jnp.concatenate(
        [s4, jnp.zeros((NC * NS, npad2), _I32)], axis=1)
    src2 = s4p.reshape(NC * NS * IRT2, K2)
    d4 = dst.reshape(NC * NS, EPT2)
    d4p = jnp.concatenate(
        [d4, jnp.full((NC * NS, npad2), N, _I32)], axis=1)
    dst2 = d4p.reshape(NC * NS * IRT2, K2)
    core_off = jnp.repeat(jnp.arange(NC, dtype=_I32) * N, NS * IRT2)
    src2 = src2 + core_off[:, None]

    zblk = jnp.zeros((ZROWS, FW), _F32)

    degp = _degree_call(edge2)

    M1, n_o, n_i = _tc0_call(degp.T, X, W1)

    agg1 = _agg_call(M1.reshape(2 * N, FW), src1, dst1, zblk,
                     k=K1, irt=IRT1, rb=RB1, nchunk=NCH1)
    M2 = _tc1_call(agg1.reshape(2, N, FW), n_i, n_o,
                   b1.reshape(2, 1, FW), W2.reshape(2, FW, FW))

    agg2 = _agg_call(M2.reshape(2 * N, FW), src2, dst2, zblk,
                     k=K2, irt=IRT2, rb=RB2, nchunk=NCH2)
    M3 = _tc2_call(agg2.reshape(2, N, FW), n_i, n_o, b2.reshape(1, FW), W3)

    agg3 = _agg_call(M3.reshape(2 * N, FW), src2, dst2, zblk,
                     k=K2, irt=IRT2, rb=RB2, nchunk=NCH2)
    return _tc3_call(agg3.reshape(2, N, FW), n_i, b3.reshape(1, 64), S.T)
